# Initial kernel scaffold; baseline (speedup 1.0000x reference)
import jax
import jax.numpy as jnp
from jax.experimental import pallas as pl


def kernel(feats, neighbors_index, neighbors_kernel_index, neighbors_dst, W1, b1, W2, b2, W3, b3, W4, b4):
    raise NotImplementedError("write your pallas kernel here")


# speedup vs baseline: 3.5507x; 3.5507x over previous
import functools

import jax
import jax.numpy as jnp
from jax import lax
from jax.experimental import pallas as pl
from jax.experimental.pallas import tpu as pltpu
from jax.experimental.pallas import tpu_sc as plsc

N_NODES = 10000
N_EDGES = 320000
KSIZE = 55
C = 128

NC, NS = 2, 16
NW = NC * NS
NPT = 320
NPAD = NW * NPT
CH = 128
NCHUNK = N_EDGES // CH


def _mm(x, w):
    nb = 10
    bn = N_NODES // nb

    def body(x_ref, w_ref, y_ref):
        y_ref[0] = jnp.dot(x_ref[...], w_ref[0], preferred_element_type=jnp.float32)

    return pl.pallas_call(
        body,
        grid=(nb, KSIZE),
        in_specs=[
            pl.BlockSpec((bn, C), lambda i, k: (i, 0)),
            pl.BlockSpec((1, C, C), lambda i, k: (k, 0, 0)),
        ],
        out_specs=pl.BlockSpec((1, bn, C), lambda i, k: (k, i, 0)),
        out_shape=jax.ShapeDtypeStruct((KSIZE, N_NODES, C), jnp.float32),
    )(x, w)


def _act(s, b):
    nb = 10
    bn = N_NODES // nb

    def body(s_ref, b_ref, o_ref):
        o_ref[...] = jnp.maximum(s_ref[...] + b_ref[0], 0.0)

    return pl.pallas_call(
        body,
        grid=(nb,),
        in_specs=[
            pl.BlockSpec((bn, C), lambda i: (i, 0)),
            pl.BlockSpec((1, C), lambda i: (0, 0)),
        ],
        out_specs=pl.BlockSpec((bn, C), lambda i: (i, 0)),
        out_shape=jax.ShapeDtypeStruct((N_NODES, C), jnp.float32),
    )(s, b.reshape(1, C))


def _sc_aggregate(y_flat, idx2, dst2, meta):
    mesh = plsc.VectorSubcoreMesh(core_axis_name="c", subcore_axis_name="s")
    WIN = NPT + 8

    @functools.partial(
        pl.kernel,
        out_type=jax.ShapeDtypeStruct((NPAD, C), jnp.float32),
        mesh=mesh,
        scratch_types=[
            pltpu.VMEM((16,), jnp.int32),
            pltpu.VMEM((CH,), jnp.int32),
            pltpu.VMEM((CH,), jnp.int32),
            pltpu.VMEM((CH,), jnp.int32),
            pltpu.VMEM((CH, C), jnp.float32),
            pltpu.VMEM_SHARED((NS * WIN, C), jnp.float32),
            pltpu.SemaphoreType.DMA,
        ],
    )
    def agg(y_hbm, idx_hbm, dst_hbm, meta_hbm, out_hbm,
            metav, idxv, dstv, ldstv, rows, win, sem):
        sid = lax.axis_index("s")
        t = sid * NC + lax.axis_index("c")
        pltpu.sync_copy(meta_hbm.at[t], metav)
        mv = metav[...]
        c0 = mv[0]
        nch = mv[1]
        base = pl.multiple_of(t * NPT, 8)
        wbase = pl.multiple_of(sid * WIN, 8)

        def zbody(i, carry):
            for j in range(8):
                rows[i, pl.ds(j * 16, 16)] = jnp.zeros((16,), jnp.float32)
            return carry

        lax.fori_loop(0, CH, zbody, 0)
        pltpu.sync_copy(rows, win.at[pl.ds(wbase, CH)])
        pltpu.sync_copy(rows, win.at[pl.ds(wbase + CH, CH)])
        pltpu.sync_copy(rows.at[pl.ds(0, WIN - 2 * CH)],
                        win.at[pl.ds(wbase + 2 * CH, WIN - 2 * CH)])

        def cbody(i, carry):
            c = c0 + i
            pltpu.sync_copy(idx_hbm.at[c], idxv)
            pltpu.sync_copy(dst_hbm.at[c], dstv)
            pltpu.async_copy(y_hbm.at[idxv], rows, sem).wait()
            for j in range(CH // 16):
                d = dstv[pl.ds(j * 16, 16)]
                l = d - base
                inb = (l >= 0) & (l < NPT)
                ldstv[pl.ds(j * 16, 16)] = jnp.where(inb, l, NPT) + wbase
            pltpu.sync_copy(rows, win.at[ldstv], add=True)
            return carry

        lax.fori_loop(0, nch, cbody, 0)
        pltpu.sync_copy(win.at[pl.ds(wbase, NPT)], out_hbm.at[pl.ds(base, NPT)])

    return agg(y_flat, idx2, dst2, meta)


def kernel(feats, neighbors_index, neighbors_kernel_index, neighbors_dst,
           W1, b1, W2, b2, W3, b3, W4, b4):
    nbr = neighbors_index.astype(jnp.int32)
    knl = neighbors_kernel_index.astype(jnp.int32)
    dst = neighbors_dst.astype(jnp.int32)

    flat = (knl * N_NODES + nbr).reshape(NCHUNK, CH)
    dst2 = dst.reshape(NCHUNK, CH)

    tgt = (jnp.arange(NW + 1) * NPT).astype(jnp.int32)
    bounds = jnp.searchsorted(dst, tgt).astype(jnp.int32)
    c0 = bounds[:-1] // CH
    c1 = (bounds[1:] + CH - 1) // CH
    meta = jnp.zeros((NW, 16), jnp.int32)
    meta = meta.at[:, 0].set(c0)
    meta = meta.at[:, 1].set(c1 - c0)

    x = feats
    for w, b in ((W1, b1), (W2, b2), (W3, b3), (W4, b4)):
        y = _mm(x, w)
        s = _sc_aggregate(y.reshape(KSIZE * N_NODES, C), flat, dst2, meta)
        x = _act(s[:N_NODES], b)
    return x


# speedup vs baseline: 4.0731x; 1.1471x over previous
import functools

import jax
import jax.numpy as jnp
from jax import lax
from jax.experimental import pallas as pl
from jax.experimental.pallas import tpu as pltpu
from jax.experimental.pallas import tpu_sc as plsc

N_NODES = 10000
N_EDGES = 320000
KSIZE = 55
C = 128

NC, NS = 2, 16
NW = NC * NS
NPT = 320
NPAD = NW * NPT
CH = 128
NCHUNK = N_EDGES // CH
Q = 4


def _mm(x, w):
    nb = 10
    bn = N_NODES // nb

    def body(x_ref, w_ref, y_ref):
        y_ref[0] = jnp.dot(x_ref[...], w_ref[0], preferred_element_type=jnp.float32)

    return pl.pallas_call(
        body,
        grid=(nb, KSIZE),
        in_specs=[
            pl.BlockSpec((bn, C), lambda i, k: (i, 0)),
            pl.BlockSpec((1, C, C), lambda i, k: (k, 0, 0)),
        ],
        out_specs=pl.BlockSpec((1, bn, C), lambda i, k: (k, i, 0)),
        out_shape=jax.ShapeDtypeStruct((KSIZE, N_NODES, C), jnp.float32),
    )(x, w)


def _act(s, b):
    nb = 10
    bn = N_NODES // nb

    def body(s_ref, b_ref, o_ref):
        o_ref[...] = jnp.maximum(s_ref[...] + b_ref[0], 0.0)

    return pl.pallas_call(
        body,
        grid=(nb,),
        in_specs=[
            pl.BlockSpec((bn, C), lambda i: (i, 0)),
            pl.BlockSpec((1, C), lambda i: (0, 0)),
        ],
        out_specs=pl.BlockSpec((bn, C), lambda i: (i, 0)),
        out_shape=jax.ShapeDtypeStruct((N_NODES, C), jnp.float32),
    )(s, b.reshape(1, C))


def _sc_aggregate(y_flat, idx2, dst2, meta):
    mesh = plsc.VectorSubcoreMesh(core_axis_name="c", subcore_axis_name="s")
    WIN = NPT + 8

    @functools.partial(
        pl.kernel,
        out_type=jax.ShapeDtypeStruct((NPAD, C), jnp.float32),
        mesh=mesh,
        scratch_types=[
            pltpu.VMEM((16,), jnp.int32),
            pltpu.VMEM((Q, CH), jnp.int32),
            pltpu.VMEM((Q, CH), jnp.int32),
            pltpu.VMEM((CH,), jnp.int32),
            [pltpu.VMEM((CH, C), jnp.float32) for _ in range(Q)],
            pltpu.VMEM_SHARED((NS * WIN, C), jnp.float32),
            pltpu.SemaphoreType.DMA,
            [pltpu.SemaphoreType.DMA for _ in range(Q)],
        ],
    )
    def agg(y_hbm, idx_hbm, dst_hbm, meta_hbm, out_hbm,
            metav, idxq, dstq, ldstv, rows, win, semi, semg):
        sid = lax.axis_index("s")
        t = sid * NC + lax.axis_index("c")
        pltpu.sync_copy(meta_hbm.at[t], metav)
        mv = metav[...]
        q0 = mv[0]
        nq = mv[1]
        base = pl.multiple_of(t * NPT, 8)
        wbase = pl.multiple_of(sid * WIN, 8)

        def zbody(i, carry):
            for j in range(8):
                rows[0][i, pl.ds(j * 16, 16)] = jnp.zeros((16,), jnp.float32)
            return carry

        lax.fori_loop(0, CH, zbody, 0)
        pltpu.sync_copy(rows[0], win.at[pl.ds(wbase, CH)])
        pltpu.sync_copy(rows[0], win.at[pl.ds(wbase + CH, CH)])
        pltpu.sync_copy(rows[0].at[pl.ds(0, WIN - 2 * CH)],
                        win.at[pl.ds(wbase + 2 * CH, WIN - 2 * CH)])

        def qbody(q, carry):
            cq = (q0 + q) * Q
            fetches = []
            for b in range(Q):
                fetches.append(pltpu.async_copy(idx_hbm.at[cq + b], idxq.at[b], semi))
                fetches.append(pltpu.async_copy(dst_hbm.at[cq + b], dstq.at[b], semi))
            for f in fetches:
                f.wait()
            gathers = [
                pltpu.async_copy(y_hbm.at[idxq.at[b]], rows[b], semg[b])
                for b in range(Q)
            ]
            for b in range(Q):
                gathers[b].wait()
                for j in range(CH // 16):
                    d = dstq[b, pl.ds(j * 16, 16)]
                    l = d - base
                    inb = (l >= 0) & (l < NPT)
                    ldstv[pl.ds(j * 16, 16)] = jnp.where(inb, l, NPT) + wbase
                pltpu.sync_copy(rows[b], win.at[ldstv], add=True)
            return carry

        lax.fori_loop(0, nq, qbody, 0)
        pltpu.sync_copy(win.at[pl.ds(wbase, NPT)], out_hbm.at[pl.ds(base, NPT)])

    return agg(y_flat, idx2, dst2, meta)


def kernel(feats, neighbors_index, neighbors_kernel_index, neighbors_dst,
           W1, b1, W2, b2, W3, b3, W4, b4):
    nbr = neighbors_index.astype(jnp.int32)
    knl = neighbors_kernel_index.astype(jnp.int32)
    dst = neighbors_dst.astype(jnp.int32)

    flat = (knl * N_NODES + nbr).reshape(NCHUNK, CH)
    dst2 = dst.reshape(NCHUNK, CH)

    tgt = (jnp.arange(NW + 1) * NPT).astype(jnp.int32)
    bounds = jnp.searchsorted(dst, tgt).astype(jnp.int32)
    q0 = bounds[:-1] // (Q * CH)
    q1 = (bounds[1:] + Q * CH - 1) // (Q * CH)
    meta = jnp.zeros((NW, 16), jnp.int32)
    meta = meta.at[:, 0].set(q0)
    meta = meta.at[:, 1].set(q1 - q0)

    x = feats
    for w, b in ((W1, b1), (W2, b2), (W3, b3), (W4, b4)):
        y = _mm(x, w)
        s = _sc_aggregate(y.reshape(KSIZE * N_NODES, C), flat, dst2, meta)
        x = _act(s[:N_NODES], b)
    return x


# speedup vs baseline: 4.0880x; 1.0037x over previous
import functools

import jax
import jax.numpy as jnp
from jax import lax
from jax.experimental import pallas as pl
from jax.experimental.pallas import tpu as pltpu
from jax.experimental.pallas import tpu_sc as plsc

N_NODES = 10000
N_EDGES = 320000
KSIZE = 55
C = 128

NC, NS = 2, 16
NW = NC * NS
NPT = 320
NPAD = NW * NPT
CH = 128
Q = 5
NCHUNK = -(-N_EDGES // CH // Q) * Q


def _mm(x, w):
    nb = 10
    bn = N_NODES // nb

    def body(x_ref, w_ref, y_ref):
        y_ref[0] = jnp.dot(x_ref[...], w_ref[0], preferred_element_type=jnp.float32)

    return pl.pallas_call(
        body,
        grid=(nb, KSIZE),
        in_specs=[
            pl.BlockSpec((bn, C), lambda i, k: (i, 0)),
            pl.BlockSpec((1, C, C), lambda i, k: (k, 0, 0)),
        ],
        out_specs=pl.BlockSpec((1, bn, C), lambda i, k: (k, i, 0)),
        out_shape=jax.ShapeDtypeStruct((KSIZE, N_NODES, C), jnp.float32),
    )(x, w)


def _act(s, b):
    nb = 10
    bn = N_NODES // nb

    def body(s_ref, b_ref, o_ref):
        o_ref[...] = jnp.maximum(s_ref[...] + b_ref[0], 0.0)

    return pl.pallas_call(
        body,
        grid=(nb,),
        in_specs=[
            pl.BlockSpec((bn, C), lambda i: (i, 0)),
            pl.BlockSpec((1, C), lambda i: (0, 0)),
        ],
        out_specs=pl.BlockSpec((bn, C), lambda i: (i, 0)),
        out_shape=jax.ShapeDtypeStruct((N_NODES, C), jnp.float32),
    )(s, b.reshape(1, C))


def _sc_aggregate(y_flat, idx2, dst2, meta):
    mesh = plsc.VectorSubcoreMesh(core_axis_name="c", subcore_axis_name="s")
    WIN = NPT + 16

    @functools.partial(
        pl.kernel,
        out_type=jax.ShapeDtypeStruct((NPAD, C), jnp.float32),
        mesh=mesh,
        scratch_types=[
            pltpu.VMEM((16,), jnp.int32),
            pltpu.VMEM((Q, CH), jnp.int32),
            pltpu.VMEM((Q, CH), jnp.int32),
            pltpu.VMEM((Q, CH), jnp.int32),
            [pltpu.VMEM((CH, C), jnp.float32) for _ in range(Q)],
            pltpu.VMEM_SHARED((NS * WIN, C), jnp.float32),
            pltpu.SemaphoreType.DMA,
            [pltpu.SemaphoreType.DMA for _ in range(Q)],
            [pltpu.SemaphoreType.DMA for _ in range(Q)],
        ],
    )
    def agg(y_hbm, idx_hbm, dst_hbm, meta_hbm, out_hbm,
            metav, idxq, dstq, ldstq, rows, win, semi, semg, sems):
        sid = lax.axis_index("s")
        t = sid * NC + lax.axis_index("c")
        pltpu.sync_copy(meta_hbm.at[t], metav)
        mv = metav[...]
        q0 = mv[0]
        nq = mv[1]
        base = pl.multiple_of(t * NPT, 16)
        wbase = pl.multiple_of(sid * WIN, 16)

        def zbody(i, carry):
            for j in range(8):
                rows[0][i, pl.ds(j * 16, 16)] = jnp.zeros((16,), jnp.float32)
            return carry

        lax.fori_loop(0, CH, zbody, 0)
        pltpu.sync_copy(rows[0], win.at[pl.ds(wbase, CH)])
        pltpu.sync_copy(rows[0], win.at[pl.ds(wbase + CH, CH)])
        pltpu.sync_copy(rows[0].at[pl.ds(0, WIN - 2 * CH)],
                        win.at[pl.ds(wbase + 2 * CH, WIN - 2 * CH)])

        def qbody(q, carry):
            cq = (q0 + q) * Q
            fetches = []
            for b in range(Q):
                fetches.append(pltpu.async_copy(idx_hbm.at[cq + b], idxq.at[b], semi))
                fetches.append(pltpu.async_copy(dst_hbm.at[cq + b], dstq.at[b], semi))
            for f in fetches:
                f.wait()
            gathers = [
                pltpu.async_copy(y_hbm.at[idxq.at[b]], rows[b], semg[b])
                for b in range(Q)
            ]
            scatters = []
            for b in range(Q):
                for j in range(CH // 16):
                    d = dstq[b, pl.ds(j * 16, 16)]
                    l = d - base
                    inb = (l >= 0) & (l < NPT)
                    ldstq[b, pl.ds(j * 16, 16)] = jnp.where(inb, l, NPT) + wbase
                gathers[b].wait()
                scatters.append(pltpu.async_copy(
                    rows[b], win.at[ldstq.at[b]], sems[b], add=True))
            for sc in scatters:
                sc.wait()
            return carry

        lax.fori_loop(0, nq, qbody, 0)
        pltpu.sync_copy(win.at[pl.ds(wbase, NPT)], out_hbm.at[pl.ds(base, NPT)])

    return agg(y_flat, idx2, dst2, meta)


def kernel(feats, neighbors_index, neighbors_kernel_index, neighbors_dst,
           W1, b1, W2, b2, W3, b3, W4, b4):
    nbr = neighbors_index.astype(jnp.int32)
    knl = neighbors_kernel_index.astype(jnp.int32)
    dst = neighbors_dst.astype(jnp.int32)

    npad_e = NCHUNK * CH - N_EDGES
    flat = jnp.pad(knl * N_NODES + nbr, (0, npad_e)).reshape(NCHUNK, CH)
    dst2 = jnp.pad(dst, (0, npad_e), constant_values=N_NODES).reshape(NCHUNK, CH)

    tgt = (jnp.arange(NW + 1) * NPT).astype(jnp.int32)
    bounds = jnp.searchsorted(dst, tgt).astype(jnp.int32)
    q0 = bounds[:-1] // (Q * CH)
    q1 = (bounds[1:] + Q * CH - 1) // (Q * CH)
    meta = jnp.zeros((NW, 16), jnp.int32)
    meta = meta.at[:, 0].set(q0)
    meta = meta.at[:, 1].set(q1 - q0)

    x = feats
    for w, b in ((W1, b1), (W2, b2), (W3, b3), (W4, b4)):
        y = _mm(x, w)
        s = _sc_aggregate(y.reshape(KSIZE * N_NODES, C), flat, dst2, meta)
        x = _act(s[:N_NODES], b)
    return x


# speedup vs baseline: 5.7191x; 1.3990x over previous
import functools

import jax
import jax.numpy as jnp
from jax import lax
from jax.experimental import pallas as pl
from jax.experimental.pallas import tpu as pltpu
from jax.experimental.pallas import tpu_sc as plsc

N_NODES = 10000
N_EDGES = 320000
KSIZE = 55
C = 128

NC, NS = 2, 16
NW = NC * NS
NPT = 320
NPAD = NW * NPT
CH = 128
Q = 5
NCHUNK = -(-N_EDGES // CH // Q) * Q


def _mm(x, w):
    nb = 5
    bn = N_NODES // nb

    def body(x_ref, w_ref, y_ref):
        y_ref[0] = jnp.dot(x_ref[...], w_ref[0], preferred_element_type=jnp.float32)

    return pl.pallas_call(
        body,
        grid=(nb, KSIZE),
        in_specs=[
            pl.BlockSpec((bn, C), lambda i, k: (i, 0)),
            pl.BlockSpec((1, C, C), lambda i, k: (k, 0, 0)),
        ],
        out_specs=pl.BlockSpec((1, bn, C), lambda i, k: (k, i, 0)),
        out_shape=jax.ShapeDtypeStruct((KSIZE, N_NODES, C), jnp.float32),
    )(x, w)


def _act(s, b):
    nb = 10
    bn = N_NODES // nb

    def body(s_ref, b_ref, o_ref):
        o_ref[...] = jnp.maximum(s_ref[...] + b_ref[0], 0.0)

    return pl.pallas_call(
        body,
        grid=(nb,),
        in_specs=[
            pl.BlockSpec((bn, C), lambda i: (i, 0)),
            pl.BlockSpec((1, C), lambda i: (0, 0)),
        ],
        out_specs=pl.BlockSpec((bn, C), lambda i: (i, 0)),
        out_shape=jax.ShapeDtypeStruct((N_NODES, C), jnp.float32),
    )(s, b.reshape(1, C))


def _sc_aggregate(y_flat, idx2, dst2, meta):
    mesh = plsc.VectorSubcoreMesh(core_axis_name="c", subcore_axis_name="s")
    WIN = NPT + 16

    @functools.partial(
        pl.kernel,
        out_type=jax.ShapeDtypeStruct((NPAD, C), jnp.float32),
        mesh=mesh,
        scratch_types=[
            pltpu.VMEM((16,), jnp.int32),
            pltpu.VMEM((Q, CH), jnp.int32),
            pltpu.VMEM((Q, CH), jnp.int32),
            pltpu.VMEM((Q, CH), jnp.int32),
            [pltpu.VMEM((CH, C), jnp.float32) for _ in range(Q)],
            pltpu.VMEM_SHARED((NS * WIN, C), jnp.float32),
            pltpu.SemaphoreType.DMA,
            [pltpu.SemaphoreType.DMA for _ in range(Q)],
            [pltpu.SemaphoreType.DMA for _ in range(Q)],
        ],
    )
    def agg(y_hbm, idx_hbm, dst_hbm, meta_hbm, out_hbm,
            metav, idxq, dstq, ldstq, rows, win, semi, semg, sems):
        sid = lax.axis_index("s")
        t = sid * NC + lax.axis_index("c")
        pltpu.sync_copy(meta_hbm.at[t], metav)
        mv = metav[...]
        q0 = mv[0]
        nq = mv[1]
        base = pl.multiple_of(t * NPT, 16)
        wbase = pl.multiple_of(sid * WIN, 16)

        def zbody(i, carry):
            for j in range(8):
                rows[0][i, pl.ds(j * 16, 16)] = jnp.zeros((16,), jnp.float32)
            return carry

        lax.fori_loop(0, CH, zbody, 0)
        pltpu.sync_copy(rows[0], win.at[pl.ds(wbase, CH)])
        pltpu.sync_copy(rows[0], win.at[pl.ds(wbase + CH, CH)])
        pltpu.sync_copy(rows[0].at[pl.ds(0, WIN - 2 * CH)],
                        win.at[pl.ds(wbase + 2 * CH, WIN - 2 * CH)])

        def qbody(q, carry):
            cq = (q0 + q) * Q
            fetches = []
            for b in range(Q):
                fetches.append(pltpu.async_copy(idx_hbm.at[cq + b], idxq.at[b], semi))
                fetches.append(pltpu.async_copy(dst_hbm.at[cq + b], dstq.at[b], semi))
            for f in fetches:
                f.wait()
            gathers = [
                pltpu.async_copy(y_hbm.at[idxq.at[b]], rows[b], semg[b])
                for b in range(Q)
            ]
            scatters = []
            for b in range(Q):
                for j in range(CH // 16):
                    d = dstq[b, pl.ds(j * 16, 16)]
                    l = d - base
                    inb = (l >= 0) & (l < NPT)
                    ldstq[b, pl.ds(j * 16, 16)] = jnp.where(inb, l, NPT) + wbase
                gathers[b].wait()
                scatters.append(pltpu.async_copy(
                    rows[b], win.at[ldstq.at[b]], sems[b], add=True))
            for sc in scatters:
                sc.wait()
            return carry

        lax.fori_loop(0, nq, qbody, 0)
        pltpu.sync_copy(win.at[pl.ds(wbase, NPT)], out_hbm.at[pl.ds(base, NPT)])

    return agg(y_flat, idx2, dst2, meta)


def kernel(feats, neighbors_index, neighbors_kernel_index, neighbors_dst,
           W1, b1, W2, b2, W3, b3, W4, b4):
    nbr = neighbors_index.astype(jnp.int32)
    knl = neighbors_kernel_index.astype(jnp.int32)
    dst = neighbors_dst.astype(jnp.int32)

    npad_e = NCHUNK * CH - N_EDGES
    flat = jnp.pad(knl * N_NODES + nbr, (0, npad_e)).reshape(NCHUNK, CH)
    dst2 = jnp.pad(dst, (0, npad_e), constant_values=N_NODES).reshape(NCHUNK, CH)

    tgt = (jnp.arange(NW + 1) * NPT).astype(jnp.int32)
    bounds = jnp.searchsorted(dst, tgt).astype(jnp.int32)
    q0 = bounds[:-1] // (Q * CH)
    q1 = (bounds[1:] + Q * CH - 1) // (Q * CH)
    meta = jnp.zeros((NW, 16), jnp.int32)
    meta = meta.at[:, 0].set(q0)
    meta = meta.at[:, 1].set(q1 - q0)

    x = feats
    for w, b in ((W1, b1), (W2, b2), (W3, b3), (W4, b4)):
        y = _mm(x, w)
        s = _sc_aggregate(y.reshape(KSIZE * N_NODES, C), flat, dst2, meta)
        x = _act(s[:N_NODES], b)
    return x


# speedup vs baseline: 5.8111x; 1.0161x over previous
import functools

import jax
import jax.numpy as jnp
from jax import lax
from jax.experimental import pallas as pl
from jax.experimental.pallas import tpu as pltpu
from jax.experimental.pallas import tpu_sc as plsc

N_NODES = 10000
N_EDGES = 320000
KSIZE = 55
C = 128

NC, NS = 2, 16
NW = NC * NS
NPT = 320
NPAD = NW * NPT
CH = 128
Q = 5
NCHUNK = -(-N_EDGES // CH // Q) * Q


def _mm(x, w, b=None, nb=5):
    bn = N_NODES // nb

    def body(x_ref, b_ref, w_ref, y_ref, xs):
        @pl.when(pl.program_id(1) == 0)
        def _():
            if b is None:
                xs[...] = x_ref[...]
            else:
                xs[...] = jnp.maximum(x_ref[...] + b_ref[0], 0.0)
        y_ref[0] = jnp.dot(xs[...], w_ref[0], preferred_element_type=jnp.float32)

    bias = (b if b is not None else jnp.zeros((C,), jnp.float32)).reshape(1, C)
    return pl.pallas_call(
        body,
        grid=(nb, KSIZE),
        in_specs=[
            pl.BlockSpec((bn, C), lambda i, k: (i, 0)),
            pl.BlockSpec((1, C), lambda i, k: (0, 0)),
            pl.BlockSpec((1, C, C), lambda i, k: (k, 0, 0)),
        ],
        out_specs=pl.BlockSpec((1, bn, C), lambda i, k: (k, i, 0)),
        out_shape=jax.ShapeDtypeStruct((KSIZE, N_NODES, C), jnp.float32),
        scratch_shapes=[pltpu.VMEM((bn, C), jnp.float32)],
    )(x, bias, w)


def _act(s, b):
    nb = 10
    bn = N_NODES // nb

    def body(s_ref, b_ref, o_ref):
        o_ref[...] = jnp.maximum(s_ref[...] + b_ref[0], 0.0)

    return pl.pallas_call(
        body,
        grid=(nb,),
        in_specs=[
            pl.BlockSpec((bn, C), lambda i: (i, 0)),
            pl.BlockSpec((1, C), lambda i: (0, 0)),
        ],
        out_specs=pl.BlockSpec((bn, C), lambda i: (i, 0)),
        out_shape=jax.ShapeDtypeStruct((N_NODES, C), jnp.float32),
    )(s, b.reshape(1, C))


def _sc_aggregate(y_flat, idx2, dst2, meta):
    mesh = plsc.VectorSubcoreMesh(core_axis_name="c", subcore_axis_name="s")
    WIN = NPT + 16

    @functools.partial(
        pl.kernel,
        out_type=jax.ShapeDtypeStruct((NPAD, C), jnp.float32),
        mesh=mesh,
        scratch_types=[
            pltpu.VMEM((16,), jnp.int32),
            pltpu.VMEM((Q, CH), jnp.int32),
            pltpu.VMEM((Q, CH), jnp.int32),
            pltpu.VMEM((Q, CH), jnp.int32),
            [pltpu.VMEM((CH, C), jnp.float32) for _ in range(Q)],
            pltpu.VMEM_SHARED((NS * WIN, C), jnp.float32),
            pltpu.SemaphoreType.DMA,
            [pltpu.SemaphoreType.DMA for _ in range(Q)],
            [pltpu.SemaphoreType.DMA for _ in range(Q)],
        ],
    )
    def agg(y_hbm, idx_hbm, dst_hbm, meta_hbm, out_hbm,
            metav, idxq, dstq, ldstq, rows, win, semi, semg, sems):
        sid = lax.axis_index("s")
        t = sid * NC + lax.axis_index("c")
        pltpu.sync_copy(meta_hbm.at[t], metav)
        mv = metav[...]
        q0 = mv[0]
        nq = mv[1]
        base = pl.multiple_of(t * NPT, 16)
        wbase = pl.multiple_of(sid * WIN, 16)

        def zbody(i, carry):
            for j in range(8):
                rows[0][i, pl.ds(j * 16, 16)] = jnp.zeros((16,), jnp.float32)
            return carry

        lax.fori_loop(0, CH, zbody, 0)
        pltpu.sync_copy(rows[0], win.at[pl.ds(wbase, CH)])
        pltpu.sync_copy(rows[0], win.at[pl.ds(wbase + CH, CH)])
        pltpu.sync_copy(rows[0].at[pl.ds(0, WIN - 2 * CH)],
                        win.at[pl.ds(wbase + 2 * CH, WIN - 2 * CH)])

        def qbody(q, carry):
            cq = (q0 + q) * Q
            fetches = []
            for b in range(Q):
                fetches.append(pltpu.async_copy(idx_hbm.at[cq + b], idxq.at[b], semi))
                fetches.append(pltpu.async_copy(dst_hbm.at[cq + b], dstq.at[b], semi))
            for f in fetches:
                f.wait()
            gathers = [
                pltpu.async_copy(y_hbm.at[idxq.at[b]], rows[b], semg[b])
                for b in range(Q)
            ]
            scatters = []
            for b in range(Q):
                for j in range(CH // 16):
                    d = dstq[b, pl.ds(j * 16, 16)]
                    l = d - base
                    inb = (l >= 0) & (l < NPT)
                    ldstq[b, pl.ds(j * 16, 16)] = jnp.where(inb, l, NPT) + wbase
                gathers[b].wait()
                scatters.append(pltpu.async_copy(
                    rows[b], win.at[ldstq.at[b]], sems[b], add=True))
            for sc in scatters:
                sc.wait()
            return carry

        lax.fori_loop(0, nq, qbody, 0)
        pltpu.sync_copy(win.at[pl.ds(wbase, NPT)], out_hbm.at[pl.ds(base, NPT)])

    return agg(y_flat, idx2, dst2, meta)


def kernel(feats, neighbors_index, neighbors_kernel_index, neighbors_dst,
           W1, b1, W2, b2, W3, b3, W4, b4):
    nbr = neighbors_index.astype(jnp.int32)
    knl = neighbors_kernel_index.astype(jnp.int32)
    dst = neighbors_dst.astype(jnp.int32)

    npad_e = NCHUNK * CH - N_EDGES
    flat = jnp.pad(knl * N_NODES + nbr, (0, npad_e)).reshape(NCHUNK, CH)
    dst2 = jnp.pad(dst, (0, npad_e), constant_values=N_NODES).reshape(NCHUNK, CH)

    tgt = (jnp.arange(NW + 1) * NPT).astype(jnp.int32)
    bounds = jnp.searchsorted(dst, tgt).astype(jnp.int32)
    q0 = bounds[:-1] // (Q * CH)
    q1 = (bounds[1:] + Q * CH - 1) // (Q * CH)
    meta = jnp.zeros((NW, 16), jnp.int32)
    meta = meta.at[:, 0].set(q0)
    meta = meta.at[:, 1].set(q1 - q0)

    y = _mm(feats, W1)
    s = _sc_aggregate(y.reshape(KSIZE * N_NODES, C), flat, dst2, meta)
    for w, b in ((W2, b2), (W3, b3), (W4, b4)):
        y = _mm(s[:N_NODES], w, b)
        s = _sc_aggregate(y.reshape(KSIZE * N_NODES, C), flat, dst2, meta)
    return _act(s[:N_NODES], b4)


# speedup vs baseline: 7.3457x; 1.2641x over previous
import functools

import jax
import jax.numpy as jnp
from jax import lax
from jax.experimental import pallas as pl
from jax.experimental.pallas import tpu as pltpu
from jax.experimental.pallas import tpu_sc as plsc

N_NODES = 10000
N_EDGES = 320000
KSIZE = 55
C = 128

NC, NS = 2, 16
NW = NC * NS
NPT = 320
NPAD = NW * NPT
CH = 128
Q = 5
NCHUNK = -(-N_EDGES // CH // Q) * Q


def _mm(x, w, b=None, nb=2):
    bn = N_NODES // nb

    def body(x_ref, b_ref, w_ref, y_ref, xs):
        @pl.when(pl.program_id(1) == 0)
        def _():
            if b is None:
                xs[...] = x_ref[...]
            else:
                xs[...] = jnp.maximum(x_ref[...] + b_ref[0], 0.0)
        y_ref[0] = jnp.dot(xs[...], w_ref[0], preferred_element_type=jnp.float32)

    bias = (b if b is not None else jnp.zeros((C,), jnp.float32)).reshape(1, C)
    return pl.pallas_call(
        body,
        grid=(nb, KSIZE),
        in_specs=[
            pl.BlockSpec((bn, C), lambda i, k: (i, 0)),
            pl.BlockSpec((1, C), lambda i, k: (0, 0)),
            pl.BlockSpec((1, C, C), lambda i, k: (k, 0, 0)),
        ],
        out_specs=pl.BlockSpec((1, bn, C), lambda i, k: (k, i, 0)),
        out_shape=jax.ShapeDtypeStruct((KSIZE, N_NODES, C), jnp.float32),
        scratch_shapes=[pltpu.VMEM((bn, C), jnp.float32)],
    )(x, bias, w)


def _act(s, b):
    nb = 10
    bn = N_NODES // nb

    def body(s_ref, b_ref, o_ref):
        o_ref[...] = jnp.maximum(s_ref[...] + b_ref[0], 0.0)

    return pl.pallas_call(
        body,
        grid=(nb,),
        in_specs=[
            pl.BlockSpec((bn, C), lambda i: (i, 0)),
            pl.BlockSpec((1, C), lambda i: (0, 0)),
        ],
        out_specs=pl.BlockSpec((bn, C), lambda i: (i, 0)),
        out_shape=jax.ShapeDtypeStruct((N_NODES, C), jnp.float32),
    )(s, b.reshape(1, C))


def _sc_aggregate(y_flat, idx2, dst2, meta):
    mesh = plsc.VectorSubcoreMesh(core_axis_name="c", subcore_axis_name="s")
    WIN = NPT + 16

    @functools.partial(
        pl.kernel,
        out_type=jax.ShapeDtypeStruct((NPAD, C), jnp.float32),
        mesh=mesh,
        scratch_types=[
            pltpu.VMEM((16,), jnp.int32),
            pltpu.VMEM((Q, CH), jnp.int32),
            pltpu.VMEM((Q, CH), jnp.int32),
            pltpu.VMEM((Q, CH), jnp.int32),
            [pltpu.VMEM((CH, C), jnp.float32) for _ in range(Q)],
            pltpu.VMEM_SHARED((NS * WIN, C), jnp.float32),
            pltpu.SemaphoreType.DMA,
            [pltpu.SemaphoreType.DMA for _ in range(Q)],
            [pltpu.SemaphoreType.DMA for _ in range(Q)],
        ],
    )
    def agg(y_hbm, idx_hbm, dst_hbm, meta_hbm, out_hbm,
            metav, idxq, dstq, ldstq, rows, win, semi, semg, sems):
        sid = lax.axis_index("s")
        t = sid * NC + lax.axis_index("c")
        pltpu.sync_copy(meta_hbm.at[t], metav)
        mv = metav[...]
        q0 = mv[0]
        nq = mv[1]
        base = pl.multiple_of(t * NPT, 16)
        wbase = pl.multiple_of(sid * WIN, 16)

        def zbody(i, carry):
            for j in range(8):
                rows[0][i, pl.ds(j * 16, 16)] = jnp.zeros((16,), jnp.float32)
            return carry

        lax.fori_loop(0, CH, zbody, 0)
        pltpu.sync_copy(rows[0], win.at[pl.ds(wbase, CH)])
        pltpu.sync_copy(rows[0], win.at[pl.ds(wbase + CH, CH)])
        pltpu.sync_copy(rows[0].at[pl.ds(0, WIN - 2 * CH)],
                        win.at[pl.ds(wbase + 2 * CH, WIN - 2 * CH)])

        def qbody(q, carry):
            cq = (q0 + q) * Q
            fetches = []
            for b in range(Q):
                fetches.append(pltpu.async_copy(idx_hbm.at[cq + b], idxq.at[b], semi))
                fetches.append(pltpu.async_copy(dst_hbm.at[cq + b], dstq.at[b], semi))
            for f in fetches:
                f.wait()
            gathers = [
                pltpu.async_copy(y_hbm.at[idxq.at[b]], rows[b], semg[b])
                for b in range(Q)
            ]
            scatters = []
            for b in range(Q):
                for j in range(CH // 16):
                    d = dstq[b, pl.ds(j * 16, 16)]
                    l = d - base
                    inb = (l >= 0) & (l < NPT)
                    ldstq[b, pl.ds(j * 16, 16)] = jnp.where(inb, l, NPT) + wbase
                gathers[b].wait()
                scatters.append(pltpu.async_copy(
                    rows[b], win.at[ldstq.at[b]], sems[b], add=True))
            for sc in scatters:
                sc.wait()
            return carry

        lax.fori_loop(0, nq, qbody, 0)
        pltpu.sync_copy(win.at[pl.ds(wbase, NPT)], out_hbm.at[pl.ds(base, NPT)])

    return agg(y_flat, idx2, dst2, meta)


def kernel(feats, neighbors_index, neighbors_kernel_index, neighbors_dst,
           W1, b1, W2, b2, W3, b3, W4, b4):
    nbr = neighbors_index.astype(jnp.int32)
    knl = neighbors_kernel_index.astype(jnp.int32)
    dst = neighbors_dst.astype(jnp.int32)

    npad_e = NCHUNK * CH - N_EDGES
    flat = jnp.pad(knl * N_NODES + nbr, (0, npad_e)).reshape(NCHUNK, CH)
    dst2 = jnp.pad(dst, (0, npad_e), constant_values=N_NODES).reshape(NCHUNK, CH)

    tgt = (jnp.arange(NW + 1) * NPT).astype(jnp.int32)
    bounds = jnp.searchsorted(dst, tgt).astype(jnp.int32)
    q0 = bounds[:-1] // (Q * CH)
    q1 = (bounds[1:] + Q * CH - 1) // (Q * CH)
    meta = jnp.zeros((NW, 16), jnp.int32)
    meta = meta.at[:, 0].set(q0)
    meta = meta.at[:, 1].set(q1 - q0)

    y = _mm(feats, W1)
    s = _sc_aggregate(y.reshape(KSIZE * N_NODES, C), flat, dst2, meta)
    for w, b in ((W2, b2), (W3, b3), (W4, b4)):
        y = _mm(s[:N_NODES], w, b)
        s = _sc_aggregate(y.reshape(KSIZE * N_NODES, C), flat, dst2, meta)
    return _act(s[:N_NODES], b4)


# speedup vs baseline: 8.5783x; 1.1678x over previous
import functools

import jax
import jax.numpy as jnp
from jax import lax
from jax.experimental import pallas as pl
from jax.experimental.pallas import tpu as pltpu
from jax.experimental.pallas import tpu_sc as plsc

N_NODES = 10000
N_EDGES = 320000
KSIZE = 55
C = 128

NC, NS = 2, 16
NW = NC * NS
NPT = 320
NPAD = NW * NPT
CH = 128
Q = 5
NCHUNK = -(-N_EDGES // CH // Q) * Q


def _mm(x, w, b=None, nb=1):
    bn = N_NODES // nb

    def body(x_ref, b_ref, w_ref, y_ref, xs):
        @pl.when(pl.program_id(1) == 0)
        def _():
            if b is None:
                xs[...] = x_ref[...]
            else:
                xs[...] = jnp.maximum(x_ref[...] + b_ref[0], 0.0)
        y_ref[0] = jnp.dot(xs[...], w_ref[0], preferred_element_type=jnp.float32)

    bias = (b if b is not None else jnp.zeros((C,), jnp.float32)).reshape(1, C)
    return pl.pallas_call(
        body,
        grid=(nb, KSIZE),
        in_specs=[
            pl.BlockSpec((bn, C), lambda i, k: (i, 0)),
            pl.BlockSpec((1, C), lambda i, k: (0, 0)),
            pl.BlockSpec((1, C, C), lambda i, k: (k, 0, 0)),
        ],
        out_specs=pl.BlockSpec((1, bn, C), lambda i, k: (k, i, 0)),
        out_shape=jax.ShapeDtypeStruct((KSIZE, N_NODES, C), jnp.float32),
        scratch_shapes=[pltpu.VMEM((bn, C), jnp.float32)],
    )(x, bias, w)


def _act(s, b):
    nb = 10
    bn = N_NODES // nb

    def body(s_ref, b_ref, o_ref):
        o_ref[...] = jnp.maximum(s_ref[...] + b_ref[0], 0.0)

    return pl.pallas_call(
        body,
        grid=(nb,),
        in_specs=[
            pl.BlockSpec((bn, C), lambda i: (i, 0)),
            pl.BlockSpec((1, C), lambda i: (0, 0)),
        ],
        out_specs=pl.BlockSpec((bn, C), lambda i: (i, 0)),
        out_shape=jax.ShapeDtypeStruct((N_NODES, C), jnp.float32),
    )(s, b.reshape(1, C))


def _sc_aggregate(y_flat, idx2, dst2, meta):
    mesh = plsc.VectorSubcoreMesh(core_axis_name="c", subcore_axis_name="s")
    WIN = NPT + 16

    @functools.partial(
        pl.kernel,
        out_type=jax.ShapeDtypeStruct((NPAD, C), jnp.float32),
        mesh=mesh,
        scratch_types=[
            pltpu.VMEM((16,), jnp.int32),
            pltpu.VMEM((Q, CH), jnp.int32),
            pltpu.VMEM((Q, CH), jnp.int32),
            pltpu.VMEM((Q, CH), jnp.int32),
            [pltpu.VMEM((CH, C), jnp.float32) for _ in range(Q)],
            pltpu.VMEM_SHARED((NS * WIN, C), jnp.float32),
            pltpu.SemaphoreType.DMA,
            [pltpu.SemaphoreType.DMA for _ in range(Q)],
            [pltpu.SemaphoreType.DMA for _ in range(Q)],
        ],
    )
    def agg(y_hbm, idx_hbm, dst_hbm, meta_hbm, out_hbm,
            metav, idxq, dstq, ldstq, rows, win, semi, semg, sems):
        sid = lax.axis_index("s")
        t = sid * NC + lax.axis_index("c")
        pltpu.sync_copy(meta_hbm.at[t], metav)
        mv = metav[...]
        q0 = mv[0]
        nq = mv[1]
        base = pl.multiple_of(t * NPT, 16)
        wbase = pl.multiple_of(sid * WIN, 16)

        def zbody(i, carry):
            for j in range(8):
                rows[0][i, pl.ds(j * 16, 16)] = jnp.zeros((16,), jnp.float32)
            return carry

        lax.fori_loop(0, CH, zbody, 0)
        pltpu.sync_copy(rows[0], win.at[pl.ds(wbase, CH)])
        pltpu.sync_copy(rows[0], win.at[pl.ds(wbase + CH, CH)])
        pltpu.sync_copy(rows[0].at[pl.ds(0, WIN - 2 * CH)],
                        win.at[pl.ds(wbase + 2 * CH, WIN - 2 * CH)])

        def qbody(q, carry):
            cq = (q0 + q) * Q
            fetches = []
            for b in range(Q):
                fetches.append(pltpu.async_copy(idx_hbm.at[cq + b], idxq.at[b], semi))
                fetches.append(pltpu.async_copy(dst_hbm.at[cq + b], dstq.at[b], semi))
            for f in fetches:
                f.wait()
            gathers = [
                pltpu.async_copy(y_hbm.at[idxq.at[b]], rows[b], semg[b])
                for b in range(Q)
            ]
            scatters = []
            for b in range(Q):
                for j in range(CH // 16):
                    d = dstq[b, pl.ds(j * 16, 16)]
                    l = d - base
                    inb = (l >= 0) & (l < NPT)
                    ldstq[b, pl.ds(j * 16, 16)] = jnp.where(inb, l, NPT) + wbase
                gathers[b].wait()
                scatters.append(pltpu.async_copy(
                    rows[b], win.at[ldstq.at[b]], sems[b], add=True))
            for sc in scatters:
                sc.wait()
            return carry

        lax.fori_loop(0, nq, qbody, 0)
        pltpu.sync_copy(win.at[pl.ds(wbase, NPT)], out_hbm.at[pl.ds(base, NPT)])

    return agg(y_flat, idx2, dst2, meta)


def kernel(feats, neighbors_index, neighbors_kernel_index, neighbors_dst,
           W1, b1, W2, b2, W3, b3, W4, b4):
    nbr = neighbors_index.astype(jnp.int32)
    knl = neighbors_kernel_index.astype(jnp.int32)
    dst = neighbors_dst.astype(jnp.int32)

    npad_e = NCHUNK * CH - N_EDGES
    flat = jnp.pad(knl * N_NODES + nbr, (0, npad_e)).reshape(NCHUNK, CH)
    dst2 = jnp.pad(dst, (0, npad_e), constant_values=N_NODES).reshape(NCHUNK, CH)

    tgt = (jnp.arange(NW + 1) * NPT).astype(jnp.int32)
    bounds = jnp.searchsorted(dst, tgt).astype(jnp.int32)
    q0 = bounds[:-1] // (Q * CH)
    q1 = (bounds[1:] + Q * CH - 1) // (Q * CH)
    meta = jnp.zeros((NW, 16), jnp.int32)
    meta = meta.at[:, 0].set(q0)
    meta = meta.at[:, 1].set(q1 - q0)

    y = _mm(feats, W1)
    s = _sc_aggregate(y.reshape(KSIZE * N_NODES, C), flat, dst2, meta)
    for w, b in ((W2, b2), (W3, b3), (W4, b4)):
        y = _mm(s[:N_NODES], w, b)
        s = _sc_aggregate(y.reshape(KSIZE * N_NODES, C), flat, dst2, meta)
    return _act(s[:N_NODES], b4)


# speedup vs baseline: 9.1061x; 1.0615x over previous
import functools

import jax
import jax.numpy as jnp
from jax import lax
from jax.experimental import pallas as pl
from jax.experimental.pallas import tpu as pltpu
from jax.experimental.pallas import tpu_sc as plsc

N_NODES = 10000
N_EDGES = 320000
KSIZE = 55
C = 128

NC, NS = 2, 16
NW = NC * NS
NPT = 320
NPAD = NW * NPT
CH = 128
Q = 5
NCHUNK = -(-N_EDGES // CH // Q) * Q


def _mm(x, w, b=None, nb=1):
    bn = N_NODES // nb

    def body(x_ref, b_ref, w_ref, y_ref, xs):
        @pl.when(pl.program_id(1) == 0)
        def _():
            if b is None:
                xs[...] = x_ref[...]
            else:
                xs[...] = jnp.maximum(x_ref[...] + b_ref[0], 0.0)
        y_ref[0] = jnp.dot(xs[...], w_ref[0], preferred_element_type=jnp.float32)

    bias = (b if b is not None else jnp.zeros((C,), jnp.float32)).reshape(1, C)
    return pl.pallas_call(
        body,
        grid=(nb, KSIZE),
        in_specs=[
            pl.BlockSpec((bn, C), lambda i, k: (i, 0)),
            pl.BlockSpec((1, C), lambda i, k: (0, 0)),
            pl.BlockSpec((1, C, C), lambda i, k: (k, 0, 0)),
        ],
        out_specs=pl.BlockSpec((1, bn, C), lambda i, k: (k, i, 0)),
        out_shape=jax.ShapeDtypeStruct((KSIZE, N_NODES, C), jnp.float32),
        scratch_shapes=[pltpu.VMEM((bn, C), jnp.float32)],
    )(x, bias, w)


def _act(s, b):
    nb = 10
    bn = N_NODES // nb

    def body(s_ref, b_ref, o_ref):
        o_ref[...] = jnp.maximum(s_ref[...] + b_ref[0], 0.0)

    return pl.pallas_call(
        body,
        grid=(nb,),
        in_specs=[
            pl.BlockSpec((bn, C), lambda i: (i, 0)),
            pl.BlockSpec((1, C), lambda i: (0, 0)),
        ],
        out_specs=pl.BlockSpec((bn, C), lambda i: (i, 0)),
        out_shape=jax.ShapeDtypeStruct((N_NODES, C), jnp.float32),
    )(s, b.reshape(1, C))


def _sc_aggregate(y_flat, idx2, dst2, meta):
    mesh = plsc.VectorSubcoreMesh(core_axis_name="c", subcore_axis_name="s")
    WIN = NPT + 16

    @functools.partial(
        pl.kernel,
        out_type=jax.ShapeDtypeStruct((NPAD, C), jnp.float32),
        mesh=mesh,
        scratch_types=[
            pltpu.VMEM((16,), jnp.int32),
            pltpu.VMEM((Q, CH), jnp.int32),
            pltpu.VMEM((Q, CH), jnp.int32),
            pltpu.VMEM((Q, CH), jnp.int32),
            [pltpu.VMEM((CH, C), jnp.float32) for _ in range(Q)],
            pltpu.VMEM_SHARED((NS * WIN, C), jnp.float32),
            pltpu.SemaphoreType.DMA,
            [pltpu.SemaphoreType.DMA for _ in range(Q)],
            [pltpu.SemaphoreType.DMA for _ in range(Q)],
        ],
    )
    def agg(y_hbm, idx_hbm, dst_hbm, meta_hbm, out_hbm,
            metav, idxq, dstq, ldstq, rows, win, semi, semg, sems):
        sid = lax.axis_index("s")
        t = sid * NC + lax.axis_index("c")
        pltpu.sync_copy(meta_hbm.at[t], metav)
        mv = metav[...]
        q0 = mv[0]
        nq = mv[1]
        base = pl.multiple_of(t * NPT, 16)
        wbase = pl.multiple_of(sid * WIN, 16)

        def zbody(i, carry):
            for j in range(8):
                rows[0][i, pl.ds(j * 16, 16)] = jnp.zeros((16,), jnp.float32)
            return carry

        lax.fori_loop(0, CH, zbody, 0)
        pltpu.sync_copy(rows[0], win.at[pl.ds(wbase, CH)])
        pltpu.sync_copy(rows[0], win.at[pl.ds(wbase + CH, CH)])
        pltpu.sync_copy(rows[0].at[pl.ds(0, WIN - 2 * CH)],
                        win.at[pl.ds(wbase + 2 * CH, WIN - 2 * CH)])

        def drain_scatters():
            for b in range(Q):
                pltpu.make_async_copy(
                    y_hbm.at[pl.ds(0, CH)], rows[b], sems[b]).wait()

        def qbody(q, carry):
            cq = (q0 + q) * Q
            fetches = []
            for b in range(Q):
                fetches.append(pltpu.async_copy(idx_hbm.at[cq + b], idxq.at[b], semi))
                fetches.append(pltpu.async_copy(dst_hbm.at[cq + b], dstq.at[b], semi))

            @pl.when(q > 0)
            def _():
                drain_scatters()

            for f in fetches:
                f.wait()
            gathers = [
                pltpu.async_copy(y_hbm.at[idxq.at[b]], rows[b], semg[b])
                for b in range(Q)
            ]
            for b in range(Q):
                for j in range(CH // 16):
                    d = dstq[b, pl.ds(j * 16, 16)]
                    l = d - base
                    inb = (l >= 0) & (l < NPT)
                    ldstq[b, pl.ds(j * 16, 16)] = jnp.where(inb, l, NPT) + wbase
                gathers[b].wait()
                pltpu.async_copy(rows[b], win.at[ldstq.at[b]], sems[b], add=True)
            return carry

        lax.fori_loop(0, nq, qbody, 0)

        @pl.when(nq > 0)
        def _():
            drain_scatters()
        pltpu.sync_copy(win.at[pl.ds(wbase, NPT)], out_hbm.at[pl.ds(base, NPT)])

    return agg(y_flat, idx2, dst2, meta)


def kernel(feats, neighbors_index, neighbors_kernel_index, neighbors_dst,
           W1, b1, W2, b2, W3, b3, W4, b4):
    nbr = neighbors_index.astype(jnp.int32)
    knl = neighbors_kernel_index.astype(jnp.int32)
    dst = neighbors_dst.astype(jnp.int32)

    npad_e = NCHUNK * CH - N_EDGES
    flat = jnp.pad(knl * N_NODES + nbr, (0, npad_e)).reshape(NCHUNK, CH)
    dst2 = jnp.pad(dst, (0, npad_e), constant_values=N_NODES).reshape(NCHUNK, CH)

    tgt = (jnp.arange(NW + 1) * NPT).astype(jnp.int32)
    bounds = jnp.searchsorted(dst, tgt).astype(jnp.int32)
    q0 = bounds[:-1] // (Q * CH)
    q1 = (bounds[1:] + Q * CH - 1) // (Q * CH)
    meta = jnp.zeros((NW, 16), jnp.int32)
    meta = meta.at[:, 0].set(q0)
    meta = meta.at[:, 1].set(q1 - q0)

    y = _mm(feats, W1)
    s = _sc_aggregate(y.reshape(KSIZE * N_NODES, C), flat, dst2, meta)
    for w, b in ((W2, b2), (W3, b3), (W4, b4)):
        y = _mm(s[:N_NODES], w, b)
        s = _sc_aggregate(y.reshape(KSIZE * N_NODES, C), flat, dst2, meta)
    return _act(s[:N_NODES], b4)
